# baseline (speedup 1.0000x reference)
import functools
import jax
import jax.numpy as jnp
from jax import lax
from jax.experimental import pallas as pl
from jax.experimental.pallas import tpu as pltpu
from jax.experimental.pallas import tpu_sc as plsc

N = 10000
F = 128
E = 320000
NC = 2
NS = 16
NW = NC * NS
CHUNK = 80
K = 128
TOTC = NW * K
EP = TOTC * CHUNK
K_C0 = 160
K_C1 = 2 * K - K_C0
NPAD = 10112
RPT = NPAD // NS


def _mesh():
    return plsc.VectorSubcoreMesh(
        core_axis_name="c", subcore_axis_name="s",
        num_cores=NC, num_subcores=NS)


@functools.cache
def _deg_sc_kernel():
    return pl.kernel(
        _deg_sc_body,
        out_type=jax.ShapeDtypeStruct((NC, NPAD, F), jnp.float32),
        mesh=_mesh(),
        scratch_types=[
            pltpu.VMEM((2, CHUNK), jnp.int32),
            pltpu.VMEM((2, CHUNK), jnp.int32),
            pltpu.VMEM((CHUNK, F), jnp.float32),
            pltpu.SemaphoreType.DMA,
            pltpu.SemaphoreType.DMA,
            pltpu.VMEM_SHARED((NPAD, F), jnp.float32),
        ],
    )


def _deg_sc_body(eidx_hbm, ones_hbm, zeros_hbm, out_hbm,
                 idx0, idx1, ones_v, semi0, semi1, acc_sh):
    cid = lax.axis_index("c")
    sid = lax.axis_index("s")
    base = (cid * NS + sid) * K
    pltpu.sync_copy(zeros_hbm.at[pl.ds(sid * RPT, RPT)],
                    acc_sh.at[pl.ds(sid * RPT, RPT)])
    pltpu.sync_copy(ones_hbm, ones_v)
    pltpu.sync_copy(eidx_hbm.at[base], idx0)
    plsc.subcore_barrier()
    pltpu.async_copy(eidx_hbm.at[base + 1], idx1, semi1)

    @pl.loop(0, K, step=2)
    def _(c):
        pltpu.sync_copy(ones_v, acc_sh.at[idx0.at[1]], add=True)

        @pl.when(c + 2 < K)
        def _():
            pltpu.async_copy(eidx_hbm.at[base + c + 2], idx0, semi0)

        pltpu.make_async_copy(eidx_hbm.at[base + c + 1], idx1, semi1).wait()
        pltpu.sync_copy(ones_v, acc_sh.at[idx1.at[1]], add=True)

        @pl.when(c + 3 < K)
        def _():
            pltpu.async_copy(eidx_hbm.at[base + c + 3], idx1, semi1)

        @pl.when(c + 2 < K)
        def _():
            pltpu.make_async_copy(eidx_hbm.at[base + c + 2], idx0, semi0).wait()

    plsc.subcore_barrier()
    pltpu.sync_copy(acc_sh.at[pl.ds(sid * RPT, RPT)],
                    out_hbm.at[cid, pl.ds(sid * RPT, RPT)])


@functools.cache
def _agg_sc_kernel():
    return pl.kernel(
        _agg_sc_body,
        out_type=jax.ShapeDtypeStruct((NC, NPAD, F), jnp.float32),
        mesh=_mesh(),
        scratch_types=[
            pltpu.VMEM((2, CHUNK), jnp.int32),
            pltpu.VMEM((2, CHUNK), jnp.int32),
            pltpu.VMEM((CHUNK, F), jnp.float32),
            pltpu.VMEM((CHUNK, F), jnp.float32),
            pltpu.SemaphoreType.DMA,
            pltpu.SemaphoreType.DMA,
            pltpu.SemaphoreType.DMA,
            pltpu.SemaphoreType.DMA,
            pltpu.VMEM_SHARED((NPAD, F), jnp.float32),
        ],
    )


def _agg_sc_body(h_hbm, eidx_hbm, zeros_hbm, out_hbm,
                 idx0, idx1, buf0, buf1, sem0, sem1, semi0, semi1, acc_sh):
    cid = lax.axis_index("c")
    sid = lax.axis_index("s")
    myk = jnp.where(cid == 0, K_C0, K_C1)
    base = jnp.where(cid == 0, sid * K_C0, NS * K_C0 + sid * K_C1)
    pltpu.sync_copy(zeros_hbm.at[pl.ds(sid * RPT, RPT)],
                    acc_sh.at[pl.ds(sid * RPT, RPT)])
    pltpu.sync_copy(eidx_hbm.at[base], idx0)
    plsc.subcore_barrier()
    pltpu.async_copy(h_hbm.at[idx0.at[0]], buf0, sem0)
    pltpu.async_copy(eidx_hbm.at[base + 1], idx1, semi1)

    @pl.loop(0, myk, step=2)
    def _(c):
        pltpu.make_async_copy(eidx_hbm.at[base + c + 1], idx1, semi1).wait()
        pltpu.async_copy(h_hbm.at[idx1.at[0]], buf1, sem1)
        pltpu.make_async_copy(h_hbm.at[idx0.at[0]], buf0, sem0).wait()
        pltpu.sync_copy(buf0, acc_sh.at[idx0.at[1]], add=True)

        @pl.when(c + 2 < myk)
        def _():
            pltpu.async_copy(eidx_hbm.at[base + c + 2], idx0, semi0)

        pltpu.make_async_copy(h_hbm.at[idx1.at[0]], buf1, sem1).wait()

        @pl.when(c + 2 < myk)
        def _():
            pltpu.make_async_copy(eidx_hbm.at[base + c + 2], idx0, semi0).wait()
            pltpu.async_copy(h_hbm.at[idx0.at[0]], buf0, sem0)

        pltpu.sync_copy(buf1, acc_sh.at[idx1.at[1]], add=True)

        @pl.when(c + 3 < myk)
        def _():
            pltpu.async_copy(eidx_hbm.at[base + c + 3], idx1, semi1)

    plsc.subcore_barrier()
    pltpu.sync_copy(acc_sh.at[pl.ds(sid * RPT, RPT)],
                    out_hbm.at[cid, pl.ds(sid * RPT, RPT)])


_RB = 1000


def _g_block(dp):
    deg = dp[0, :, 0:1] + dp[1, :, 0:1] + 1.0
    return lax.rsqrt(deg)


def _mm1_body(x_ref, w_ref, dp_ref, o_ref):
    g = _g_block(dp_ref[...])
    h = jnp.dot(x_ref[...], w_ref[...], preferred_element_type=jnp.float32)
    o_ref[...] = h * g


def _mid_body(p_ref, hp_ref, dp_ref, b_ref, w_ref, o_ref):
    g = _g_block(dp_ref[...])
    agg = p_ref[0] + p_ref[1] + hp_ref[...]
    z = jnp.maximum(g * agg + b_ref[...], 0.0)
    o_ref[...] = g * jnp.dot(z, w_ref[...], preferred_element_type=jnp.float32)


def _fin_body(p_ref, hp_ref, dp_ref, b_ref, o_ref):
    g = _g_block(dp_ref[...])
    agg = p_ref[0] + p_ref[1] + hp_ref[...]
    o_ref[...] = g * agg + b_ref[...]


def _row_spec():
    return pl.BlockSpec((_RB, F), lambda i: (i, 0))


def _p_spec():
    return pl.BlockSpec((2, _RB, F), lambda i: (0, i, 0))


def _dp_spec():
    return pl.BlockSpec((2, _RB, F), lambda i: (0, i, 0))


def _full_spec(shape):
    return pl.BlockSpec(shape, lambda i: tuple(0 for _ in shape))


def _mm1_tc(x, W1, dp):
    return pl.pallas_call(
        _mm1_body,
        grid=(N // _RB,),
        in_specs=[_row_spec(), _full_spec((F, F)), _dp_spec()],
        out_specs=_row_spec(),
        out_shape=jax.ShapeDtypeStruct((N, F), jnp.float32),
    )(x, W1, dp)


def _mid_tc(p, hp, dp, b, W2):
    return pl.pallas_call(
        _mid_body,
        grid=(N // _RB,),
        in_specs=[_p_spec(), _row_spec(), _dp_spec(),
                  _full_spec((1, F)), _full_spec((F, F))],
        out_specs=_row_spec(),
        out_shape=jax.ShapeDtypeStruct((N, F), jnp.float32),
    )(p, hp, dp, b, W2)


def _fin_tc(p, hp, dp, b):
    return pl.pallas_call(
        _fin_body,
        grid=(N // _RB,),
        in_specs=[_p_spec(), _row_spec(), _dp_spec(), _full_spec((1, F))],
        out_specs=_row_spec(),
        out_shape=jax.ShapeDtypeStruct((N, F), jnp.float32),
    )(p, hp, dp, b)


@jax.jit
def kernel(x, edge_index, W1, b1, W2, b2):
    ei = edge_index.astype(jnp.int32)
    pad = EP - E
    src_p = jnp.concatenate(
        [ei[0], jnp.zeros((pad,), jnp.int32)]).reshape(TOTC, CHUNK)
    dst_p = jnp.concatenate(
        [ei[1], jnp.full((pad,), N, jnp.int32)]).reshape(TOTC, CHUNK)
    onesF = jnp.ones((CHUNK, F), jnp.float32)
    zerosF = jnp.zeros((NPAD, F), jnp.float32)

    eidx = jnp.stack([src_p, dst_p], axis=1)

    dp = _deg_sc_kernel()(eidx, onesF, zerosF)
    h1p = _mm1_tc(x, W1, dp)
    p1 = _agg_sc_kernel()(h1p, eidx, zerosF)
    h2p = _mid_tc(p1, h1p, dp, b1.reshape(1, F), W2)
    p2 = _agg_sc_kernel()(h2p, eidx, zerosF)
    return _fin_tc(p2, h2p, dp, b2.reshape(1, F))


# speedup vs baseline: 1.8890x; 1.8890x over previous
import functools
import jax
import jax.numpy as jnp
from jax import lax
from jax.experimental import pallas as pl
from jax.experimental.pallas import tpu as pltpu
from jax.experimental.pallas import tpu_sc as plsc

N = 10000
F = 128
E = 320000
NC = 2
NS = 16
NW = NC * NS
CHUNK = 48
D = 4
K_C0 = 300
K_C1 = 120
TOTC = NS * (K_C0 + K_C1)
EP = TOTC * CHUNK
KD = TOTC // NW
NPAD = 10112
RPT = NPAD // NS


def _mesh():
    return plsc.VectorSubcoreMesh(
        core_axis_name="c", subcore_axis_name="s",
        num_cores=NC, num_subcores=NS)


@functools.cache
def _deg_sc_kernel():
    return pl.kernel(
        _deg_sc_body,
        out_type=jax.ShapeDtypeStruct((NC, NPAD, F), jnp.float32),
        mesh=_mesh(),
        scratch_types=[
            pltpu.VMEM((2, CHUNK), jnp.int32),
            pltpu.VMEM((2, CHUNK), jnp.int32),
            pltpu.VMEM((CHUNK, F), jnp.float32),
            pltpu.SemaphoreType.DMA,
            pltpu.SemaphoreType.DMA,
            pltpu.VMEM_SHARED((NPAD, F), jnp.float32),
        ],
    )


def _deg_sc_body(eidx_hbm, ones_hbm, zeros_hbm, out_hbm,
                 idx0, idx1, ones_v, semi0, semi1, acc_sh):
    cid = lax.axis_index("c")
    sid = lax.axis_index("s")
    base = (cid * NS + sid) * KD
    pltpu.sync_copy(zeros_hbm.at[pl.ds(sid * RPT, RPT)],
                    acc_sh.at[pl.ds(sid * RPT, RPT)])
    pltpu.sync_copy(ones_hbm, ones_v)
    pltpu.sync_copy(eidx_hbm.at[base], idx0)
    plsc.subcore_barrier()
    pltpu.async_copy(eidx_hbm.at[base + 1], idx1, semi1)

    @pl.loop(0, KD, step=2)
    def _(c):
        pltpu.sync_copy(ones_v, acc_sh.at[idx0.at[1]], add=True)

        @pl.when(c + 2 < KD)
        def _():
            pltpu.async_copy(eidx_hbm.at[base + c + 2], idx0, semi0)

        pltpu.make_async_copy(eidx_hbm.at[base + c + 1], idx1, semi1).wait()
        pltpu.sync_copy(ones_v, acc_sh.at[idx1.at[1]], add=True)

        @pl.when(c + 3 < KD)
        def _():
            pltpu.async_copy(eidx_hbm.at[base + c + 3], idx1, semi1)

        @pl.when(c + 2 < KD)
        def _():
            pltpu.make_async_copy(eidx_hbm.at[base + c + 2], idx0, semi0).wait()

    plsc.subcore_barrier()
    pltpu.sync_copy(acc_sh.at[pl.ds(sid * RPT, RPT)],
                    out_hbm.at[cid, pl.ds(sid * RPT, RPT)])


@functools.cache
def _agg_sc_kernel():
    return pl.kernel(
        _agg_sc_body,
        out_type=jax.ShapeDtypeStruct((NC, NPAD, F), jnp.float32),
        mesh=_mesh(),
        scratch_types=(
            [pltpu.VMEM((2, CHUNK), jnp.int32) for _ in range(D)] +
            [pltpu.VMEM((CHUNK, F), jnp.float32) for _ in range(D)] +
            [pltpu.SemaphoreType.DMA for _ in range(2 * D)] +
            [pltpu.VMEM_SHARED((NPAD, F), jnp.float32)]
        ),
    )


def _agg_sc_body(h_hbm, eidx_hbm, zeros_hbm, out_hbm, *scratch):
    idxs = scratch[0:D]
    bufs = scratch[D:2 * D]
    gsem = scratch[2 * D:3 * D]
    isem = scratch[3 * D:4 * D]
    acc_sh = scratch[4 * D]
    cid = lax.axis_index("c")
    sid = lax.axis_index("s")
    myk = jnp.where(cid == 0, K_C0, K_C1)
    base = jnp.where(cid == 0, sid * K_C0, NS * K_C0 + sid * K_C1)
    for j in range(D):
        pltpu.async_copy(eidx_hbm.at[base + j], idxs[j], isem[j])
    pltpu.sync_copy(zeros_hbm.at[pl.ds(sid * RPT, RPT)],
                    acc_sh.at[pl.ds(sid * RPT, RPT)])
    plsc.subcore_barrier()

    @pl.loop(0, myk, step=D)
    def _(c):
        for j in range(D):
            pltpu.make_async_copy(
                eidx_hbm.at[base + c + j], idxs[j], isem[j]).wait()
            pltpu.async_copy(h_hbm.at[idxs[j].at[0]], bufs[j], gsem[j])
        for j in range(D):
            pltpu.make_async_copy(
                h_hbm.at[idxs[j].at[0]], bufs[j], gsem[j]).wait()
            pltpu.sync_copy(bufs[j], acc_sh.at[idxs[j].at[1]], add=True)

            @pl.when(c + j + D < myk)
            def _():
                pltpu.async_copy(
                    eidx_hbm.at[base + c + j + D], idxs[j], isem[j])

    plsc.subcore_barrier()
    pltpu.sync_copy(acc_sh.at[pl.ds(sid * RPT, RPT)],
                    out_hbm.at[cid, pl.ds(sid * RPT, RPT)])


_RB = 1000


def _g_block(dp):
    deg = dp[0, :, 0:1] + dp[1, :, 0:1] + 1.0
    return lax.rsqrt(deg)


def _mm1_body(x_ref, w_ref, dp_ref, o_ref):
    g = _g_block(dp_ref[...])
    h = jnp.dot(x_ref[...], w_ref[...], preferred_element_type=jnp.float32)
    o_ref[...] = h * g


def _mid_body(p_ref, hp_ref, dp_ref, b_ref, w_ref, o_ref):
    g = _g_block(dp_ref[...])
    agg = p_ref[0] + p_ref[1] + hp_ref[...]
    z = jnp.maximum(g * agg + b_ref[...], 0.0)
    o_ref[...] = g * jnp.dot(z, w_ref[...], preferred_element_type=jnp.float32)


def _fin_body(p_ref, hp_ref, dp_ref, b_ref, o_ref):
    g = _g_block(dp_ref[...])
    agg = p_ref[0] + p_ref[1] + hp_ref[...]
    o_ref[...] = g * agg + b_ref[...]


def _row_spec():
    return pl.BlockSpec((_RB, F), lambda i: (i, 0))


def _p_spec():
    return pl.BlockSpec((2, _RB, F), lambda i: (0, i, 0))


def _dp_spec():
    return pl.BlockSpec((2, _RB, F), lambda i: (0, i, 0))


def _full_spec(shape):
    return pl.BlockSpec(shape, lambda i: tuple(0 for _ in shape))


def _mm1_tc(x, W1, dp):
    return pl.pallas_call(
        _mm1_body,
        grid=(N // _RB,),
        in_specs=[_row_spec(), _full_spec((F, F)), _dp_spec()],
        out_specs=_row_spec(),
        out_shape=jax.ShapeDtypeStruct((N, F), jnp.float32),
    )(x, W1, dp)


def _mid_tc(p, hp, dp, b, W2):
    return pl.pallas_call(
        _mid_body,
        grid=(N // _RB,),
        in_specs=[_p_spec(), _row_spec(), _dp_spec(),
                  _full_spec((1, F)), _full_spec((F, F))],
        out_specs=_row_spec(),
        out_shape=jax.ShapeDtypeStruct((N, F), jnp.float32),
    )(p, hp, dp, b, W2)


def _fin_tc(p, hp, dp, b):
    return pl.pallas_call(
        _fin_body,
        grid=(N // _RB,),
        in_specs=[_p_spec(), _row_spec(), _dp_spec(), _full_spec((1, F))],
        out_specs=_row_spec(),
        out_shape=jax.ShapeDtypeStruct((N, F), jnp.float32),
    )(p, hp, dp, b)


@jax.jit
def kernel(x, edge_index, W1, b1, W2, b2):
    ei = edge_index.astype(jnp.int32)
    pad = EP - E
    src_p = jnp.concatenate(
        [ei[0], jnp.zeros((pad,), jnp.int32)]).reshape(TOTC, CHUNK)
    dst_p = jnp.concatenate(
        [ei[1], jnp.full((pad,), N, jnp.int32)]).reshape(TOTC, CHUNK)
    eidx = jnp.stack([src_p, dst_p], axis=1)
    onesF = jnp.ones((CHUNK, F), jnp.float32)
    zerosF = jnp.zeros((NPAD, F), jnp.float32)

    dp = _deg_sc_kernel()(eidx, onesF, zerosF)
    h1p = _mm1_tc(x, W1, dp)
    p1 = _agg_sc_kernel()(h1p, eidx, zerosF)
    h2p = _mid_tc(p1, h1p, dp, b1.reshape(1, F), W2)
    p2 = _agg_sc_kernel()(h2p, eidx, zerosF)
    return _fin_tc(p2, h2p, dp, b2.reshape(1, F))


# speedup vs baseline: 2.0095x; 1.0638x over previous
import functools
import jax
import jax.numpy as jnp
from jax import lax
from jax.experimental import pallas as pl
from jax.experimental.pallas import tpu as pltpu
from jax.experimental.pallas import tpu_sc as plsc

N = 10000
F = 128
E = 320000
NC = 2
NS = 16
NW = NC * NS
CHUNK = 48
D = 4
K_C0 = 300
K_C1 = 120
TOTC = NS * (K_C0 + K_C1)
EP = TOTC * CHUNK
CHUNK_D = 128
TOTC_D = 2560
EP_D = TOTC_D * CHUNK_D
KD = TOTC_D // NW
NPAD = 10112
RPT = NPAD // NS


def _mesh():
    return plsc.VectorSubcoreMesh(
        core_axis_name="c", subcore_axis_name="s",
        num_cores=NC, num_subcores=NS)


@functools.cache
def _deg_sc_kernel():
    return pl.kernel(
        _deg_sc_body,
        out_type=jax.ShapeDtypeStruct((NC, NPAD, F), jnp.float32),
        mesh=_mesh(),
        scratch_types=[
            pltpu.VMEM((1, CHUNK_D), jnp.int32),
            pltpu.VMEM((1, CHUNK_D), jnp.int32),
            pltpu.VMEM((CHUNK_D, F), jnp.float32),
            pltpu.SemaphoreType.DMA,
            pltpu.SemaphoreType.DMA,
            pltpu.VMEM_SHARED((NPAD, F), jnp.float32),
        ],
    )


def _deg_sc_body(dstd_hbm, ones_hbm, zeros_hbm, out_hbm,
                 idx0, idx1, ones_v, semi0, semi1, acc_sh):
    cid = lax.axis_index("c")
    sid = lax.axis_index("s")
    base = (cid * NS + sid) * KD
    pltpu.sync_copy(zeros_hbm.at[pl.ds(sid * RPT, RPT)],
                    acc_sh.at[pl.ds(sid * RPT, RPT)])
    pltpu.sync_copy(ones_hbm, ones_v)
    pltpu.sync_copy(dstd_hbm.at[base], idx0)
    plsc.subcore_barrier()
    pltpu.async_copy(dstd_hbm.at[base + 1], idx1, semi1)

    @pl.loop(0, KD, step=2)
    def _(c):
        pltpu.sync_copy(ones_v, acc_sh.at[idx0.at[0]], add=True)

        @pl.when(c + 2 < KD)
        def _():
            pltpu.async_copy(dstd_hbm.at[base + c + 2], idx0, semi0)

        pltpu.make_async_copy(dstd_hbm.at[base + c + 1], idx1, semi1).wait()
        pltpu.sync_copy(ones_v, acc_sh.at[idx1.at[0]], add=True)

        @pl.when(c + 3 < KD)
        def _():
            pltpu.async_copy(dstd_hbm.at[base + c + 3], idx1, semi1)

        @pl.when(c + 2 < KD)
        def _():
            pltpu.make_async_copy(dstd_hbm.at[base + c + 2], idx0, semi0).wait()

    plsc.subcore_barrier()
    pltpu.sync_copy(acc_sh.at[pl.ds(sid * RPT, RPT)],
                    out_hbm.at[cid, pl.ds(sid * RPT, RPT)])


@functools.cache
def _agg_sc_kernel():
    return pl.kernel(
        _agg_sc_body,
        out_type=jax.ShapeDtypeStruct((NC, NPAD, F), jnp.float32),
        mesh=_mesh(),
        scratch_types=(
            [pltpu.VMEM((2, CHUNK), jnp.int32) for _ in range(D)] +
            [pltpu.VMEM((CHUNK, F), jnp.float32) for _ in range(D)] +
            [pltpu.SemaphoreType.DMA for _ in range(2 * D)] +
            [pltpu.VMEM_SHARED((NPAD, F), jnp.float32)]
        ),
    )


def _agg_sc_body(h_hbm, eidx_hbm, zeros_hbm, out_hbm, *scratch):
    idxs = scratch[0:D]
    bufs = scratch[D:2 * D]
    gsem = scratch[2 * D:3 * D]
    isem = scratch[3 * D:4 * D]
    acc_sh = scratch[4 * D]
    cid = lax.axis_index("c")
    sid = lax.axis_index("s")
    myk = jnp.where(cid == 0, K_C0, K_C1)
    base = jnp.where(cid == 0, sid * K_C0, NS * K_C0 + sid * K_C1)
    for j in range(D):
        pltpu.async_copy(eidx_hbm.at[base + j], idxs[j], isem[j])
    pltpu.sync_copy(zeros_hbm.at[pl.ds(sid * RPT, RPT)],
                    acc_sh.at[pl.ds(sid * RPT, RPT)])
    plsc.subcore_barrier()

    @pl.loop(0, myk, step=D)
    def _(c):
        for j in range(D):
            pltpu.make_async_copy(
                eidx_hbm.at[base + c + j], idxs[j], isem[j]).wait()
            pltpu.async_copy(h_hbm.at[idxs[j].at[0]], bufs[j], gsem[j])
        for j in range(D):
            pltpu.make_async_copy(
                h_hbm.at[idxs[j].at[0]], bufs[j], gsem[j]).wait()
            pltpu.sync_copy(bufs[j], acc_sh.at[idxs[j].at[1]], add=True)

            @pl.when(c + j + D < myk)
            def _():
                pltpu.async_copy(
                    eidx_hbm.at[base + c + j + D], idxs[j], isem[j])

    plsc.subcore_barrier()
    pltpu.sync_copy(acc_sh.at[pl.ds(sid * RPT, RPT)],
                    out_hbm.at[cid, pl.ds(sid * RPT, RPT)])


_RB = 1000


def _g_block(dp):
    deg = dp[0, :, 0:1] + dp[1, :, 0:1] + 1.0
    return lax.rsqrt(deg)


def _mm1_body(x_ref, w_ref, dp_ref, o_ref):
    g = _g_block(dp_ref[...])
    h = jnp.dot(x_ref[...], w_ref[...], preferred_element_type=jnp.float32)
    o_ref[...] = h * g


def _mid_body(p_ref, hp_ref, dp_ref, b_ref, w_ref, o_ref):
    g = _g_block(dp_ref[...])
    agg = p_ref[0] + p_ref[1] + hp_ref[...]
    z = jnp.maximum(g * agg + b_ref[...], 0.0)
    o_ref[...] = g * jnp.dot(z, w_ref[...], preferred_element_type=jnp.float32)


def _fin_body(p_ref, hp_ref, dp_ref, b_ref, o_ref):
    g = _g_block(dp_ref[...])
    agg = p_ref[0] + p_ref[1] + hp_ref[...]
    o_ref[...] = g * agg + b_ref[...]


def _row_spec():
    return pl.BlockSpec((_RB, F), lambda i: (i, 0))


def _p_spec():
    return pl.BlockSpec((2, _RB, F), lambda i: (0, i, 0))


def _dp_spec():
    return pl.BlockSpec((2, _RB, F), lambda i: (0, i, 0))


def _full_spec(shape):
    return pl.BlockSpec(shape, lambda i: tuple(0 for _ in shape))


def _mm1_tc(x, W1, dp):
    return pl.pallas_call(
        _mm1_body,
        grid=(N // _RB,),
        in_specs=[_row_spec(), _full_spec((F, F)), _dp_spec()],
        out_specs=_row_spec(),
        out_shape=jax.ShapeDtypeStruct((N, F), jnp.float32),
    )(x, W1, dp)


def _mid_tc(p, hp, dp, b, W2):
    return pl.pallas_call(
        _mid_body,
        grid=(N // _RB,),
        in_specs=[_p_spec(), _row_spec(), _dp_spec(),
                  _full_spec((1, F)), _full_spec((F, F))],
        out_specs=_row_spec(),
        out_shape=jax.ShapeDtypeStruct((N, F), jnp.float32),
    )(p, hp, dp, b, W2)


def _fin_tc(p, hp, dp, b):
    return pl.pallas_call(
        _fin_body,
        grid=(N // _RB,),
        in_specs=[_p_spec(), _row_spec(), _dp_spec(), _full_spec((1, F))],
        out_specs=_row_spec(),
        out_shape=jax.ShapeDtypeStruct((N, F), jnp.float32),
    )(p, hp, dp, b)


@jax.jit
def kernel(x, edge_index, W1, b1, W2, b2):
    ei = edge_index.astype(jnp.int32)
    pad = EP - E
    src_p = jnp.concatenate(
        [ei[0], jnp.zeros((pad,), jnp.int32)]).reshape(TOTC, CHUNK)
    dst_p = jnp.concatenate(
        [ei[1], jnp.full((pad,), N, jnp.int32)]).reshape(TOTC, CHUNK)
    eidx = jnp.stack([src_p, dst_p], axis=1)
    dst_d = jnp.concatenate(
        [ei[1], jnp.full((EP_D - E,), N, jnp.int32)]).reshape(TOTC_D, 1, CHUNK_D)
    onesD = jnp.ones((CHUNK_D, F), jnp.float32)
    zerosF = jnp.zeros((NPAD, F), jnp.float32)

    dp = _deg_sc_kernel()(dst_d, onesD, zerosF)
    h1p = _mm1_tc(x, W1, dp)
    p1 = _agg_sc_kernel()(h1p, eidx, zerosF)
    h2p = _mid_tc(p1, h1p, dp, b1.reshape(1, F), W2)
    p2 = _agg_sc_kernel()(h2p, eidx, zerosF)
    return _fin_tc(p2, h2p, dp, b2.reshape(1, F))


# speedup vs baseline: 2.0213x; 1.0059x over previous
import functools
import jax
import jax.numpy as jnp
from jax import lax
from jax.experimental import pallas as pl
from jax.experimental.pallas import tpu as pltpu
from jax.experimental.pallas import tpu_sc as plsc

N = 10000
F = 128
E = 320000
NC = 2
NS = 16
NW = NC * NS
CHUNK = 48
D = 4
K_C0 = 300
K_C1 = 120
TOTC = NS * (K_C0 + K_C1)
EP = TOTC * CHUNK
CHUNK_D = 128
TOTC_D = 2560
EP_D = TOTC_D * CHUNK_D
KD = TOTC_D // NW
NPAD = 10112
RPT = NPAD // NS


def _mesh():
    return plsc.VectorSubcoreMesh(
        core_axis_name="c", subcore_axis_name="s",
        num_cores=NC, num_subcores=NS)


@functools.cache
def _deg_sc_kernel():
    return pl.kernel(
        _deg_sc_body,
        out_type=jax.ShapeDtypeStruct((NC, NPAD, F), jnp.float32),
        mesh=_mesh(),
        scratch_types=[
            pltpu.VMEM((1, CHUNK_D), jnp.int32),
            pltpu.VMEM((1, CHUNK_D), jnp.int32),
            pltpu.VMEM((CHUNK_D, F), jnp.float32),
            pltpu.SemaphoreType.DMA,
            pltpu.SemaphoreType.DMA,
            pltpu.VMEM_SHARED((NPAD, F), jnp.float32),
        ],
    )


def _deg_sc_body(dstd_hbm, ones_hbm, zeros_hbm, out_hbm,
                 idx0, idx1, ones_v, semi0, semi1, acc_sh):
    cid = lax.axis_index("c")
    sid = lax.axis_index("s")
    base = (cid * NS + sid) * KD
    pltpu.sync_copy(zeros_hbm.at[pl.ds(sid * RPT, RPT)],
                    acc_sh.at[pl.ds(sid * RPT, RPT)])
    pltpu.sync_copy(ones_hbm, ones_v)
    pltpu.sync_copy(dstd_hbm.at[base], idx0)
    plsc.subcore_barrier()
    pltpu.async_copy(dstd_hbm.at[base + 1], idx1, semi1)

    @pl.loop(0, KD, step=2)
    def _(c):
        pltpu.sync_copy(ones_v, acc_sh.at[idx0.at[0]], add=True)

        @pl.when(c + 2 < KD)
        def _():
            pltpu.async_copy(dstd_hbm.at[base + c + 2], idx0, semi0)

        pltpu.make_async_copy(dstd_hbm.at[base + c + 1], idx1, semi1).wait()
        pltpu.sync_copy(ones_v, acc_sh.at[idx1.at[0]], add=True)

        @pl.when(c + 3 < KD)
        def _():
            pltpu.async_copy(dstd_hbm.at[base + c + 3], idx1, semi1)

        @pl.when(c + 2 < KD)
        def _():
            pltpu.make_async_copy(dstd_hbm.at[base + c + 2], idx0, semi0).wait()

    plsc.subcore_barrier()
    pltpu.sync_copy(acc_sh.at[pl.ds(sid * RPT, RPT)],
                    out_hbm.at[cid, pl.ds(sid * RPT, RPT)])


@functools.cache
def _agg_sc_kernel():
    return pl.kernel(
        _agg_sc_body,
        out_type=jax.ShapeDtypeStruct((NC, NPAD, F), jnp.float32),
        mesh=_mesh(),
        scratch_types=(
            [pltpu.VMEM((2, CHUNK), jnp.int32) for _ in range(D)] +
            [pltpu.VMEM((CHUNK, F), jnp.float32) for _ in range(D)] +
            [pltpu.SemaphoreType.DMA for _ in range(2 * D)] +
            [pltpu.VMEM_SHARED((NPAD, F), jnp.float32)]
        ),
    )


def _agg_sc_body(h_hbm, eidx_hbm, zeros_hbm, out_hbm, *scratch):
    idxs = scratch[0:D]
    bufs = scratch[D:2 * D]
    gsem = scratch[2 * D:3 * D]
    isem = scratch[3 * D:4 * D]
    acc_sh = scratch[4 * D]
    cid = lax.axis_index("c")
    sid = lax.axis_index("s")
    myk = jnp.where(cid == 0, K_C0, K_C1)
    base = jnp.where(cid == 0, sid * K_C0, NS * K_C0 + sid * K_C1)
    for j in range(D):
        pltpu.async_copy(eidx_hbm.at[:, base + j], idxs[j], isem[j])
    pltpu.sync_copy(zeros_hbm.at[pl.ds(sid * RPT, RPT)],
                    acc_sh.at[pl.ds(sid * RPT, RPT)])
    plsc.subcore_barrier()

    @pl.loop(0, myk, step=D)
    def _(c):
        for j in range(D):
            pltpu.make_async_copy(
                eidx_hbm.at[:, base + c + j], idxs[j], isem[j]).wait()
            pltpu.async_copy(h_hbm.at[idxs[j].at[0]], bufs[j], gsem[j])
        for j in range(D):
            pltpu.make_async_copy(
                h_hbm.at[idxs[j].at[0]], bufs[j], gsem[j]).wait()
            pltpu.sync_copy(bufs[j], acc_sh.at[idxs[j].at[1]], add=True)

            @pl.when(c + j + D < myk)
            def _():
                pltpu.async_copy(
                    eidx_hbm.at[:, base + c + j + D], idxs[j], isem[j])

    plsc.subcore_barrier()
    pltpu.sync_copy(acc_sh.at[pl.ds(sid * RPT, RPT)],
                    out_hbm.at[cid, pl.ds(sid * RPT, RPT)])


_RB = 1000


def _g_block(dp):
    deg = dp[0, :, 0:1] + dp[1, :, 0:1] + 1.0
    return lax.rsqrt(deg)


def _mm1_body(x_ref, w_ref, dp_ref, o_ref):
    g = _g_block(dp_ref[...])
    h = jnp.dot(x_ref[...], w_ref[...], preferred_element_type=jnp.float32)
    o_ref[...] = h * g


def _mid_body(p_ref, hp_ref, dp_ref, b_ref, w_ref, o_ref):
    g = _g_block(dp_ref[...])
    agg = p_ref[0] + p_ref[1] + hp_ref[...]
    z = jnp.maximum(g * agg + b_ref[...], 0.0)
    o_ref[...] = g * jnp.dot(z, w_ref[...], preferred_element_type=jnp.float32)


def _fin_body(p_ref, hp_ref, dp_ref, b_ref, o_ref):
    g = _g_block(dp_ref[...])
    agg = p_ref[0] + p_ref[1] + hp_ref[...]
    o_ref[...] = g * agg + b_ref[...]


def _row_spec():
    return pl.BlockSpec((_RB, F), lambda i: (i, 0))


def _p_spec():
    return pl.BlockSpec((2, _RB, F), lambda i: (0, i, 0))


def _dp_spec():
    return pl.BlockSpec((2, _RB, F), lambda i: (0, i, 0))


def _full_spec(shape):
    return pl.BlockSpec(shape, lambda i: tuple(0 for _ in shape))


def _mm1_tc(x, W1, dp):
    return pl.pallas_call(
        _mm1_body,
        grid=(N // _RB,),
        in_specs=[_row_spec(), _full_spec((F, F)), _dp_spec()],
        out_specs=_row_spec(),
        out_shape=jax.ShapeDtypeStruct((N, F), jnp.float32),
    )(x, W1, dp)


def _mid_tc(p, hp, dp, b, W2):
    return pl.pallas_call(
        _mid_body,
        grid=(N // _RB,),
        in_specs=[_p_spec(), _row_spec(), _dp_spec(),
                  _full_spec((1, F)), _full_spec((F, F))],
        out_specs=_row_spec(),
        out_shape=jax.ShapeDtypeStruct((N, F), jnp.float32),
    )(p, hp, dp, b, W2)


def _fin_tc(p, hp, dp, b):
    return pl.pallas_call(
        _fin_body,
        grid=(N // _RB,),
        in_specs=[_p_spec(), _row_spec(), _dp_spec(), _full_spec((1, F))],
        out_specs=_row_spec(),
        out_shape=jax.ShapeDtypeStruct((N, F), jnp.float32),
    )(p, hp, dp, b)


@jax.jit
def kernel(x, edge_index, W1, b1, W2, b2):
    ei = edge_index.astype(jnp.int32)
    pad = EP - E
    dst_d = jnp.concatenate(
        [ei[1], jnp.full((EP_D - E,), N, jnp.int32)]).reshape(TOTC_D, 1, CHUNK_D)
    padv = jnp.stack([jnp.zeros((pad,), jnp.int32), jnp.full((pad,), N, jnp.int32)])
    eidx = jnp.concatenate([ei, padv], axis=1).reshape(NC, TOTC, CHUNK)
    onesD = jnp.ones((CHUNK_D, F), jnp.float32)
    zerosF = jnp.zeros((NPAD, F), jnp.float32)

    dp = _deg_sc_kernel()(dst_d, onesD, zerosF)
    h1p = _mm1_tc(x, W1, dp)
    p1 = _agg_sc_kernel()(h1p, eidx, zerosF)
    h2p = _mid_tc(p1, h1p, dp, b1.reshape(1, F), W2)
    p2 = _agg_sc_kernel()(h2p, eidx, zerosF)
    return _fin_tc(p2, h2p, dp, b2.reshape(1, F))
